# baseline (speedup 1.0000x reference)
import jax
import jax.numpy as jnp
from jax import lax
from jax.experimental import pallas as pl
from jax.experimental.pallas import tpu as pltpu
from jax.experimental.pallas import tpu_sc as plsc

D = 1024
LANES = 16
NC, NS = 2, 16
NW = NC * NS
B_TOTAL = 4 * 2048
ROWS_PER_W = B_TOTAL // NW
CHUNK = 32
NCHUNK = ROWS_PER_W // CHUNK
VECS_PER_ROW = D // LANES


def _pe_body(x_hbm, n_hbm, emb_hbm, out_hbm, idx_v, rows_v, x_v, sem_g):
    wid = lax.axis_index("s") * NC + lax.axis_index("c")
    base = wid * ROWS_PER_W

    for c in range(NCHUNK):
        row0 = base + c * CHUNK
        pltpu.sync_copy(n_hbm.at[pl.ds(row0, CHUNK)], idx_v)
        pltpu.async_copy(emb_hbm.at[idx_v], rows_v, sem_g).wait()
        pltpu.sync_copy(x_hbm.at[pl.ds(row0, CHUNK)], x_v)

        def row_body(r, carry):
            def vec_body(j, carry2):
                col = j * LANES
                x_v[r, pl.ds(col, LANES)] = (
                    x_v[r, pl.ds(col, LANES)] + rows_v[r, pl.ds(col, LANES)])
                return carry2
            return lax.fori_loop(0, VECS_PER_ROW, vec_body, carry)
        lax.fori_loop(0, CHUNK, row_body, 0)

        pltpu.sync_copy(x_v, out_hbm.at[pl.ds(row0, CHUNK)])


@jax.jit
def kernel(x, n, emb):
    B, S, Dm = x.shape
    x2 = x.reshape(B * S, Dm)
    n1 = n.reshape(B * S).astype(jnp.int32)
    mesh = plsc.VectorSubcoreMesh(
        core_axis_name="c", subcore_axis_name="s",
        num_cores=NC, num_subcores=NS)
    run = pl.kernel(
        _pe_body,
        out_type=jax.ShapeDtypeStruct((B * S, Dm), jnp.float32),
        mesh=mesh,
        scratch_types=[
            pltpu.VMEM((CHUNK,), jnp.int32),
            pltpu.VMEM((CHUNK, Dm), jnp.float32),
            pltpu.VMEM((CHUNK, Dm), jnp.float32),
            pltpu.SemaphoreType.DMA,
        ],
    )
    out = run(x2, n1, emb)
    return out.reshape(B, S, Dm)


# speedup vs baseline: 2.3744x; 2.3744x over previous
import jax
import jax.numpy as jnp
from jax import lax
from jax.experimental import pallas as pl
from jax.experimental.pallas import tpu as pltpu
from jax.experimental.pallas import tpu_sc as plsc

D = 1024
LANES = 16
NC, NS = 2, 16
NW = NC * NS
B_TOTAL = 4 * 2048
ROWS_PER_W = B_TOTAL // NW
CHUNK = 16
NCHUNK = ROWS_PER_W // CHUNK
NSET = 3
VECS_PER_ROW = D // LANES


def _pe_body(x_hbm, n_hbm, emb_hbm, out_hbm, idx_all,
             rows0, rows1, rows2, xb0, xb1, xb2,
             sg0, sg1, sg2, sx0, sx1, sx2, so0, so1, so2):
    rows = [rows0, rows1, rows2]
    xb = [xb0, xb1, xb2]
    sg = [sg0, sg1, sg2]
    sx = [sx0, sx1, sx2]
    so = [so0, so1, so2]

    wid = lax.axis_index("s") * NC + lax.axis_index("c")
    base = wid * ROWS_PER_W
    pltpu.sync_copy(n_hbm.at[pl.ds(base, ROWS_PER_W)], idx_all)

    gfut = [None] * NSET
    xfut = [None] * NSET
    ofut = [None] * NSET

    def issue(t):
        b = t % NSET
        row0 = base + t * CHUNK
        if ofut[b] is not None:
            ofut[b].wait()
            ofut[b] = None
        gfut[b] = pltpu.async_copy(
            emb_hbm.at[idx_all.at[pl.ds(t * CHUNK, CHUNK)]], rows[b], sg[b])
        xfut[b] = pltpu.async_copy(x_hbm.at[pl.ds(row0, CHUNK)], xb[b], sx[b])

    issue(0)
    for t in range(NCHUNK):
        b = t % NSET
        if t + 1 < NCHUNK:
            issue(t + 1)
        gfut[b].wait()
        xfut[b].wait()

        def row_body(r, carry):
            @plsc.parallel_loop(0, VECS_PER_ROW, unroll=8)
            def vec_body(j):
                col = j * LANES
                xb[b][r, pl.ds(col, LANES)] = (
                    xb[b][r, pl.ds(col, LANES)] + rows[b][r, pl.ds(col, LANES)])
            return carry
        lax.fori_loop(0, CHUNK, row_body, 0)

        ofut[b] = pltpu.async_copy(
            xb[b], out_hbm.at[pl.ds(base + t * CHUNK, CHUNK)], so[b])

    for b in range(NSET):
        if ofut[b] is not None:
            ofut[b].wait()


@jax.jit
def kernel(x, n, emb):
    B, S, Dm = x.shape
    x2 = x.reshape(B * S, Dm)
    n1 = n.reshape(B * S).astype(jnp.int32)
    mesh = plsc.VectorSubcoreMesh(
        core_axis_name="c", subcore_axis_name="s",
        num_cores=NC, num_subcores=NS)
    run = pl.kernel(
        _pe_body,
        out_type=jax.ShapeDtypeStruct((B * S, Dm), jnp.float32),
        mesh=mesh,
        scratch_types=(
            [pltpu.VMEM((ROWS_PER_W,), jnp.int32)]
            + [pltpu.VMEM((CHUNK, Dm), jnp.float32) for _ in range(2 * NSET)]
            + [pltpu.SemaphoreType.DMA for _ in range(3 * NSET)]
        ),
    )
    out = run(x2, n1, emb)
    return out.reshape(B, S, Dm)
